# Initial kernel scaffold; baseline (speedup 1.0000x reference)
import jax
import jax.numpy as jnp
from jax.experimental import pallas as pl


def kernel(anchor, positive, negative):
    raise NotImplementedError("write your pallas kernel here")


# speedup vs baseline: 1.4304x; 1.4304x over previous
import jax
import jax.numpy as jnp
from jax.experimental import pallas as pl
from jax.experimental.pallas import tpu as pltpu

_B, _D = 16384, 1024
_MARGIN = (0.2 + 0.5) / 2.0
_EPS = 1e-8
_K = _B // 2
_BLK = 1024
_NBLK = _B // _BLK


def _tl_kernel(a_ref, p_ref, n_ref, out_ref, dp_ref, dn_ref):
    i = pl.program_id(0)
    a = a_ref[...]
    p = p_ref[...]
    n = n_ref[...]
    aa = jnp.sum(a * a, axis=1)
    pp = jnp.sum(p * p, axis=1)
    nn = jnp.sum(n * n, axis=1)
    ap = jnp.sum(a * p, axis=1)
    an = jnp.sum(a * n, axis=1)
    na = jnp.maximum(jnp.sqrt(aa), _EPS)
    dp = 1.0 - ap / (na * jnp.maximum(jnp.sqrt(pp), _EPS))
    dn = 1.0 - an / (na * jnp.maximum(jnp.sqrt(nn), _EPS))
    dp_ref[pl.ds(i, 1), :] = dp.reshape(1, _BLK)
    dn_ref[pl.ds(i, 1), :] = dn.reshape(1, _BLK)

    @pl.when(i == _NBLK - 1)
    def _select():
        dnv = dn_ref[...]
        dpv = dp_ref[...]
        u = jax.lax.bitcast_convert_type(dnv, jnp.uint32)
        key = jnp.where((u >> 31) != 0, ~u, u | jnp.uint32(0x80000000))

        def vbody(it, pfx):
            b = (31 - it).astype(jnp.uint32)
            cand = pfx | (jnp.uint32(1) << b)
            cnt = jnp.sum(jnp.where(key >= cand, 1, 0))
            return jnp.where(cnt >= _K, cand, pfx)

        t = jax.lax.fori_loop(0, 32, vbody, jnp.uint32(0))

        gt = key > t
        eq = key == t
        need = _K - jnp.sum(jnp.where(gt, 1, 0))
        idx = (jax.lax.broadcasted_iota(jnp.int32, (_NBLK, _BLK), 0) * _BLK
               + jax.lax.broadcasted_iota(jnp.int32, (_NBLK, _BLK), 1))

        def ibody(_, lohi):
            lo, hi = lohi
            mid = (lo + hi) // 2
            g = jnp.sum(jnp.where(eq & (idx < mid), 1, 0))
            return (jnp.where(g >= need, lo, mid), jnp.where(g >= need, mid, hi))

        _, m = jax.lax.fori_loop(0, 15, ibody, (jnp.int32(0), jnp.int32(_B)))

        sel = gt | (eq & (idx < m))
        loss = jnp.maximum(dpv - dnv + _MARGIN, 0.0)
        total = jnp.sum(jnp.where(sel, loss, 0.0)) / _K
        out_ref[...] = total.reshape(1, 1)


def kernel(anchor, positive, negative):
    out = pl.pallas_call(
        _tl_kernel,
        grid=(_NBLK,),
        in_specs=[
            pl.BlockSpec((_BLK, _D), lambda i: (i, 0)),
            pl.BlockSpec((_BLK, _D), lambda i: (i, 0)),
            pl.BlockSpec((_BLK, _D), lambda i: (i, 0)),
        ],
        out_specs=pl.BlockSpec((1, 1), lambda i: (0, 0)),
        out_shape=jax.ShapeDtypeStruct((1, 1), jnp.float32),
        scratch_shapes=[
            pltpu.VMEM((_NBLK, _BLK), jnp.float32),
            pltpu.VMEM((_NBLK, _BLK), jnp.float32),
        ],
        compiler_params=pltpu.CompilerParams(
            dimension_semantics=("arbitrary",),
        ),
    )(anchor, positive, negative)
    return out[0, 0]
